# baseline (speedup 1.0000x reference)
import jax
import jax.numpy as jnp
from jax import lax
from jax.experimental import pallas as pl
from jax.experimental.pallas import tpu as pltpu
from jax.experimental.pallas import tpu_sc as plsc

B = 4096
CONT_F = 13
CATE_F = 26
CATE_FP = 32
COMB_F = 325
COMB_FP = 384
OD = 40
CD = 10
CDP = 16
H = 100

NC, NS = 2, 16
NW = NC * NS

CHUNK = 128
CATE_CH_W = (B // NW) * CATE_FP // CHUNK
COMB_CH_W = (B // NW) * COMB_FP // CHUNK
CATE_WIN = 4
COMB_WIN = 16
CATE_NWIN = CATE_CH_W // CATE_WIN
COMB_NWIN = COMB_CH_W // COMB_WIN


def _sc_gather_body(cates_hbm, combs_hbm, cate_tab, comb_tab,
                    cate_out, comb_out,
                    cate_idx_v, comb_idx_v, cate_buf, comb_buf, sem):
    wid = lax.axis_index("s") * NC + lax.axis_index("c")

    pltpu.sync_copy(cates_hbm.at[wid], cate_idx_v)
    pltpu.sync_copy(combs_hbm.at[wid], comb_idx_v)

    def win_loop(tab, idx_v, buf, out, ch_per_w, win):
        def body(w, carry):
            handles = []
            for j in range(win):
                h = pltpu.async_copy(
                    tab.at[idx_v.at[w * win + j]], buf.at[j], sem)
                handles.append(h)
            for h in handles:
                h.wait()
            pltpu.sync_copy(buf, out.at[pl.ds(wid * ch_per_w + w * win, win)])
            return carry
        return body

    lax.fori_loop(0, CATE_NWIN,
                  win_loop(cate_tab, cate_idx_v, cate_buf, cate_out,
                           CATE_CH_W, CATE_WIN), 0, unroll=False)
    lax.fori_loop(0, COMB_NWIN,
                  win_loop(comb_tab, comb_idx_v, comb_buf, comb_out,
                           COMB_CH_W, COMB_WIN), 0, unroll=False)


def _sc_gather(cates_w, combs_w, cate_table_b, comb_table_b):
    mesh = plsc.VectorSubcoreMesh(core_axis_name="c", subcore_axis_name="s")
    run = pl.kernel(
        _sc_gather_body,
        out_type=(
            jax.ShapeDtypeStruct((NW * CATE_CH_W, CHUNK, OD), jnp.float32),
            jax.ShapeDtypeStruct((NW * COMB_CH_W, CHUNK, CDP), jnp.float32),
        ),
        mesh=mesh,
        scratch_types=[
            pltpu.VMEM((CATE_CH_W, CHUNK), jnp.int32),
            pltpu.VMEM((COMB_CH_W, CHUNK), jnp.int32),
            pltpu.VMEM((CATE_WIN, CHUNK, OD), jnp.float32),
            pltpu.VMEM((COMB_WIN, CHUNK, CDP), jnp.float32),
            pltpu.SemaphoreType.DMA,
        ],
        compiler_params=pltpu.CompilerParams(use_tc_tiling_on_sc=False),
    )
    return run(cates_w, combs_w, cate_table_b, comb_table_b)


BT = 256


def _tc_mlp_body(conts_ref, xc_ref, xb_ref, e_ref, w1cs_ref, w1cat_ref,
                 w1comb_ref, b1_ref, g1_ref, be1_ref, w2_ref, b2_ref, g2_ref,
                 be2_ref, wo_ref, bo_ref, out_ref):
    f32 = jnp.float32
    ce = jnp.dot(conts_ref[...], e_ref[...], preferred_element_type=f32)
    h = jnp.dot(ce, w1cs_ref[...], preferred_element_type=f32)
    h += jnp.dot(xc_ref[...], w1cat_ref[...], preferred_element_type=f32)
    h += jnp.dot(xb_ref[...], w1comb_ref[...], preferred_element_type=f32)
    h += b1_ref[...]

    def layernorm(x, g, b):
        m = jnp.mean(x, axis=-1, keepdims=True)
        v = jnp.mean((x - m) * (x - m), axis=-1, keepdims=True)
        return (x - m) * lax.rsqrt(v + 1e-5) * g + b

    h = jnp.maximum(layernorm(h, g1_ref[...], be1_ref[...]), 0.0)
    h = jnp.dot(h, w2_ref[...], preferred_element_type=f32) + b2_ref[...]
    h = jnp.maximum(layernorm(h, g2_ref[...], be2_ref[...]), 0.0)
    logit = jnp.dot(h, wo_ref[...], preferred_element_type=f32) + bo_ref[...]
    out_ref[...] = 1.0 / (1.0 + jnp.exp(-logit))


def _tc_mlp(conts, xc, xb, e_mat, w1cs, w1cat, w1comb, b1, g1, be1,
            w2t, b2, g2, be2, wot, bo):
    full = lambda shape: pl.BlockSpec(shape, lambda i: (0, 0))
    tile = lambda w: pl.BlockSpec((BT, w), lambda i: (i, 0))
    return pl.pallas_call(
        _tc_mlp_body,
        grid=(B // BT,),
        in_specs=[
            tile(CONT_F),
            tile(CATE_FP * OD),
            tile(COMB_FP * CDP),
            full((CONT_F, CONT_F * OD)),
            full((CONT_F * OD, H)),
            full((CATE_FP * OD, H)),
            full((COMB_FP * CDP, H)),
            full((1, H)), full((1, H)), full((1, H)),
            full((H, H)),
            full((1, H)), full((1, H)), full((1, H)),
            full((H, 1)),
            full((1, 1)),
        ],
        out_specs=pl.BlockSpec((BT, 1), lambda i: (i, 0)),
        out_shape=jax.ShapeDtypeStruct((B, 1), jnp.float32),
    )(conts, xc, xb, e_mat, w1cs, w1cat, w1comb, b1, g1, be1,
      w2t, b2, g2, be2, wot, bo)


def kernel(conts, cates, combs, cate_table, comb_table,
           W1, b1, g1, be1, W2, b2, g2, be2, Wo, bo):
    cates_w = jnp.concatenate(
        [cates, jnp.zeros((B, CATE_FP - CATE_F), jnp.int32)], axis=1)
    cates_w = cates_w.reshape(NW, CATE_CH_W, CHUNK)
    combs_w = jnp.concatenate(
        [combs, jnp.zeros((B, COMB_FP - COMB_F), jnp.int32)], axis=1)
    combs_w = combs_w.reshape(NW, COMB_CH_W, CHUNK)

    comb_p = jnp.pad(comb_table, ((0, 0), (0, CDP - CD)))
    cate_rows, comb_rows = _sc_gather(cates_w, combs_w, cate_table, comb_p)
    xc = cate_rows.reshape(B, CATE_FP * OD)
    xb = comb_rows.reshape(B, COMB_FP * CDP)

    n_cont = CONT_F * OD
    c13_flat = cate_table[:CONT_F].reshape(n_cont)
    e_mat = jnp.repeat(jnp.eye(CONT_F, dtype=jnp.float32), OD, axis=1)
    w1cs = W1[:, :n_cont].T * c13_flat[:, None]
    w1cat = W1[:, n_cont:n_cont + CATE_F * OD].T
    w1cat = jnp.concatenate(
        [w1cat, jnp.zeros(((CATE_FP - CATE_F) * OD, H), jnp.float32)],
        axis=0)
    w1comb = W1[:, n_cont + CATE_F * OD:].T.reshape(COMB_F, CD, H)
    w1comb = jnp.concatenate(
        [w1comb, jnp.zeros((COMB_F, CDP - CD, H), jnp.float32)], axis=1)
    w1comb = jnp.concatenate(
        [w1comb, jnp.zeros((COMB_FP - COMB_F, CDP, H), jnp.float32)], axis=0)
    w1comb = w1comb.reshape(COMB_FP * CDP, H)

    out = _tc_mlp(conts, xc, xb, e_mat, w1cs, w1cat, w1comb,
                  b1.reshape(1, H), g1.reshape(1, H), be1.reshape(1, H),
                  W2.T, b2.reshape(1, H), g2.reshape(1, H), be2.reshape(1, H),
                  Wo.T, bo.reshape(1, 1))
    return out


# speedup vs baseline: 1.5426x; 1.5426x over previous
import jax
import jax.numpy as jnp
from jax import lax
from jax.experimental import pallas as pl
from jax.experimental.pallas import tpu as pltpu
from jax.experimental.pallas import tpu_sc as plsc

B = 4096
CONT_F = 13
CATE_F = 26
CATE_FP = 32
COMB_F = 325
COMB_FP = 328
OD = 40
CD = 10
CDP = 16
H = 100

NC, NS = 2, 16
NW = NC * NS

CHUNK = 128
CATE_CH_W = (B // NW) * CATE_FP // CHUNK
COMB_CH_W = (B // NW) * COMB_FP // CHUNK
CATE_WIN = 2
COMB_WIN = 8
CATE_NWIN = CATE_CH_W // CATE_WIN
COMB_NWIN = COMB_CH_W // COMB_WIN


def _sc_gather_body(cates_hbm, combs_hbm, cate_tab, comb_tab,
                    cate_out, comb_out,
                    cate_idx_v, comb_idx_v, cate_buf, comb_buf, sem, osem):
    wid = lax.axis_index("s") * NC + lax.axis_index("c")

    pltpu.sync_copy(cates_hbm.at[wid], cate_idx_v)
    pltpu.sync_copy(combs_hbm.at[wid], comb_idx_v)

    def win_loop(tab, idx_v, buf, out, ch_per_w, win, nwin, gsem, osem):
        obase = wid * ch_per_w

        def drain_gathers(parity):
            for j in range(win):
                pltpu.make_async_copy(
                    tab.at[idx_v.at[0]], buf.at[parity, j], gsem).wait()

        def body(w, carry):
            parity = lax.rem(w, 2)
            @pl.when(w >= 2)
            def _():
                pltpu.make_async_copy(
                    buf.at[parity], out.at[pl.ds(obase, win)], osem).wait()
            for j in range(win):
                pltpu.async_copy(
                    tab.at[idx_v.at[w * win + j]], buf.at[parity, j], gsem)
            @pl.when(w >= 1)
            def _():
                drain_gathers(1 - parity)
                pltpu.async_copy(
                    buf.at[1 - parity],
                    out.at[pl.ds(obase + (w - 1) * win, win)], osem)
            return carry

        lax.fori_loop(0, nwin, body, 0, unroll=False)
        last = nwin - 1
        drain_gathers(last % 2)
        pltpu.async_copy(buf.at[last % 2],
                         out.at[pl.ds(obase + last * win, win)], osem)
        for k in range(min(2, nwin)):
            pltpu.make_async_copy(
                buf.at[k], out.at[pl.ds(obase, win)], osem).wait()

    win_loop(cate_tab, cate_idx_v, cate_buf, cate_out,
             CATE_CH_W, CATE_WIN, CATE_NWIN, sem, osem)
    win_loop(comb_tab, comb_idx_v, comb_buf, comb_out,
             COMB_CH_W, COMB_WIN, COMB_NWIN, sem, osem)


def _sc_gather(cates_w, combs_w, cate_table_b, comb_table_b):
    mesh = plsc.VectorSubcoreMesh(core_axis_name="c", subcore_axis_name="s")
    run = pl.kernel(
        _sc_gather_body,
        out_type=(
            jax.ShapeDtypeStruct((NW * CATE_CH_W, CHUNK, OD), jnp.float32),
            jax.ShapeDtypeStruct((NW * COMB_CH_W, CHUNK, CDP), jnp.float32),
        ),
        mesh=mesh,
        scratch_types=[
            pltpu.VMEM((CATE_CH_W, CHUNK), jnp.int32),
            pltpu.VMEM((COMB_CH_W, CHUNK), jnp.int32),
            pltpu.VMEM((2, CATE_WIN, CHUNK, OD), jnp.float32),
            pltpu.VMEM((2, COMB_WIN, CHUNK, CDP), jnp.float32),
            pltpu.SemaphoreType.DMA,
            pltpu.SemaphoreType.DMA,
        ],
        compiler_params=pltpu.CompilerParams(use_tc_tiling_on_sc=False),
    )
    return run(cates_w, combs_w, cate_table_b, comb_table_b)


BT = 256


def _tc_mlp_body(conts_ref, xc_ref, xb_ref, e_ref, w1cs_ref, w1cat_ref,
                 w1comb_ref, b1_ref, g1_ref, be1_ref, w2_ref, b2_ref, g2_ref,
                 be2_ref, wo_ref, bo_ref, out_ref):
    f32 = jnp.float32
    ce = jnp.dot(conts_ref[...], e_ref[...], preferred_element_type=f32)
    h = jnp.dot(ce, w1cs_ref[...], preferred_element_type=f32)
    h += jnp.dot(xc_ref[...], w1cat_ref[...], preferred_element_type=f32)
    h += jnp.dot(xb_ref[...], w1comb_ref[...], preferred_element_type=f32)
    h += b1_ref[...]

    def layernorm(x, g, b):
        m = jnp.mean(x, axis=-1, keepdims=True)
        v = jnp.mean((x - m) * (x - m), axis=-1, keepdims=True)
        return (x - m) * lax.rsqrt(v + 1e-5) * g + b

    h = jnp.maximum(layernorm(h, g1_ref[...], be1_ref[...]), 0.0)
    h = jnp.dot(h, w2_ref[...], preferred_element_type=f32) + b2_ref[...]
    h = jnp.maximum(layernorm(h, g2_ref[...], be2_ref[...]), 0.0)
    logit = jnp.dot(h, wo_ref[...], preferred_element_type=f32) + bo_ref[...]
    out_ref[...] = 1.0 / (1.0 + jnp.exp(-logit))


def _tc_mlp(conts, xc, xb, e_mat, w1cs, w1cat, w1comb, b1, g1, be1,
            w2t, b2, g2, be2, wot, bo):
    full = lambda shape: pl.BlockSpec(shape, lambda i: (0, 0))
    tile = lambda w: pl.BlockSpec((BT, w), lambda i: (i, 0))
    return pl.pallas_call(
        _tc_mlp_body,
        grid=(B // BT,),
        in_specs=[
            tile(CONT_F),
            tile(CATE_FP * OD),
            tile(COMB_FP * CDP),
            full((CONT_F, CONT_F * OD)),
            full((CONT_F * OD, H)),
            full((CATE_FP * OD, H)),
            full((COMB_FP * CDP, H)),
            full((1, H)), full((1, H)), full((1, H)),
            full((H, H)),
            full((1, H)), full((1, H)), full((1, H)),
            full((H, 1)),
            full((1, 1)),
        ],
        out_specs=pl.BlockSpec((BT, 1), lambda i: (i, 0)),
        out_shape=jax.ShapeDtypeStruct((B, 1), jnp.float32),
    )(conts, xc, xb, e_mat, w1cs, w1cat, w1comb, b1, g1, be1,
      w2t, b2, g2, be2, wot, bo)


def kernel(conts, cates, combs, cate_table, comb_table,
           W1, b1, g1, be1, W2, b2, g2, be2, Wo, bo):
    cates_w = jnp.concatenate(
        [cates, jnp.zeros((B, CATE_FP - CATE_F), jnp.int32)], axis=1)
    cates_w = cates_w.reshape(NW, CATE_CH_W, CHUNK)
    combs_w = jnp.concatenate(
        [combs, jnp.zeros((B, COMB_FP - COMB_F), jnp.int32)], axis=1)
    combs_w = combs_w.reshape(NW, COMB_CH_W, CHUNK)

    comb_p = jnp.pad(comb_table, ((0, 0), (0, CDP - CD)))
    cate_rows, comb_rows = _sc_gather(cates_w, combs_w, cate_table, comb_p)
    xc = cate_rows.reshape(B, CATE_FP * OD)
    xb = comb_rows.reshape(B, COMB_FP * CDP)

    n_cont = CONT_F * OD
    c13_flat = cate_table[:CONT_F].reshape(n_cont)
    e_mat = jnp.repeat(jnp.eye(CONT_F, dtype=jnp.float32), OD, axis=1)
    w1cs = W1[:, :n_cont].T * c13_flat[:, None]
    w1cat = W1[:, n_cont:n_cont + CATE_F * OD].T
    w1cat = jnp.concatenate(
        [w1cat, jnp.zeros(((CATE_FP - CATE_F) * OD, H), jnp.float32)],
        axis=0)
    w1comb = W1[:, n_cont + CATE_F * OD:].T.reshape(COMB_F, CD, H)
    w1comb = jnp.concatenate(
        [w1comb, jnp.zeros((COMB_F, CDP - CD, H), jnp.float32)], axis=1)
    w1comb = jnp.concatenate(
        [w1comb, jnp.zeros((COMB_FP - COMB_F, CDP, H), jnp.float32)], axis=0)
    w1comb = w1comb.reshape(COMB_FP * CDP, H)

    out = _tc_mlp(conts, xc, xb, e_mat, w1cs, w1cat, w1comb,
                  b1.reshape(1, H), g1.reshape(1, H), be1.reshape(1, H),
                  W2.T, b2.reshape(1, H), g2.reshape(1, H), be2.reshape(1, H),
                  Wo.T, bo.reshape(1, 1))
    return out
